# Initial kernel scaffold; baseline (speedup 1.0000x reference)
import jax
import jax.numpy as jnp
from jax.experimental import pallas as pl


def kernel(x, edge_index, edge_weight, Wh1, bh1, Wx1, bx1, Wl1, bl1, Wg1, bg1, Wh2, bh2, Wx2, bx2, Wl2, bl2, Wg2, bg2, Wc, bc):
    raise NotImplementedError("write your pallas kernel here")


# speedup vs baseline: 2.7029x; 2.7029x over previous
import functools

import jax
import jax.numpy as jnp
from jax import lax
from jax.experimental import pallas as pl
from jax.experimental.pallas import tpu as pltpu
from jax.experimental.pallas import tpu_sc as plsc

N = 10000
E = 320000
FEAT = 128
HID = 128
NCLS = 40

NC = 2
NS = 16
L = 16
NW = NC * NS

NPAD = 10240
DEG_ROWS = NPAD // L
EPT_DEG = E // NS
EW_PER = E // NW
CH = EW_PER // L
ROWS_PER_TILE = N // NS

_MESH = plsc.VectorSubcoreMesh(
    core_axis_name="c", subcore_axis_name="s", num_cores=NC, num_subcores=NS)
_SC_PARAMS = pltpu.CompilerParams(
    needs_layout_passes=False, use_tc_tiling_on_sc=False)


def _rsqrt16(x):
    i = plsc.bitcast(x, jnp.int32)
    i = jnp.int32(0x5F3759DF) - lax.shift_right_logical(i, 1)
    y = plsc.bitcast(i, jnp.float32)
    for _ in range(3):
        y = y * (1.5 - 0.5 * x * y * y)
    return y


@functools.partial(
    pl.kernel,
    out_type=jax.ShapeDtypeStruct((NS * NPAD,), jnp.float32),
    mesh=_MESH,
    compiler_params=_SC_PARAMS,
    scratch_types=[
        pltpu.VMEM((EPT_DEG,), jnp.int32),
        pltpu.VMEM((EPT_DEG,), jnp.float32),
        pltpu.VMEM((NPAD,), jnp.float32),
    ],
)
def _deg_partial(dst_hbm, ew_hbm, degp_hbm, dstv, ewv, degv):
    c = lax.axis_index("c")
    s = lax.axis_index("s")

    @pl.when(c == 0)
    def _():
        base = s * EPT_DEG
        pltpu.sync_copy(dst_hbm.at[pl.ds(base, EPT_DEG)], dstv)
        pltpu.sync_copy(ew_hbm.at[pl.ds(base, EPT_DEG)], ewv)

        zero = jnp.zeros((L,), jnp.float32)

        def zbody(i, _):
            degv[pl.ds(i * L, L)] = zero
            return 0
        lax.fori_loop(0, NPAD // L, zbody, 0)

        def abody(i, _):
            d16 = dstv[pl.ds(i * L, L)]
            w16 = ewv[pl.ds(i * L, L)]
            plsc.addupdate_scatter(degv, [d16], w16)
            return 0
        lax.fori_loop(0, EPT_DEG // L, abody, 0)

        pltpu.sync_copy(degv, degp_hbm.at[pl.ds(s * NPAD, NPAD)])


def _tcdis_body(degp_ref, dis_ref):
    dis_ref[:] = lax.rsqrt(jnp.sum(degp_ref[:], axis=0, keepdims=True) + 1.0)


def _tcdis(degp):
    return pl.pallas_call(
        _tcdis_body,
        grid=(1,),
        in_specs=[pl.BlockSpec((NS, NPAD), lambda i: (0, 0))],
        out_specs=[pl.BlockSpec((1, NPAD), lambda i: (0, 0))],
        out_shape=[jax.ShapeDtypeStruct((1, NPAD), jnp.float32)],
    )(degp)[0].reshape(NPAD)


NRT = NPAD // NS
HQ = HID // 4
NRND = 2
EPT = E // NS
CH2 = EPT // L

@functools.partial(
    pl.kernel,
    out_type=jax.ShapeDtypeStruct((NC, NRND, NPAD, HQ), jnp.float32),
    mesh=_MESH,
    compiler_params=_SC_PARAMS,
    scratch_types=[
        pltpu.VMEM((EPT,), jnp.int32),
        pltpu.VMEM((EPT,), jnp.int32),
        pltpu.VMEM((CH2, L), jnp.int32),
        pltpu.VMEM((EPT,), jnp.float32),
        pltpu.VMEM((NPAD,), jnp.float32),
        pltpu.VMEM((L, HQ), jnp.float32),
        pltpu.VMEM((128, HQ), jnp.float32),
        pltpu.VMEM_SHARED((NPAD, HQ), jnp.float32),
        pltpu.SemaphoreType.DMA,
    ],
)
def _spmv(gr_hbm, src_hbm, dst_hbm, ew_hbm, dis_hbm, out_hbm,
          srcv, dstv1, dstv2, ewv, disv, rows, zbuf, acc, gsem):
    c = lax.axis_index("c")
    s = lax.axis_index("s")
    base_e = s * EPT

    pltpu.sync_copy(src_hbm.at[pl.ds(base_e, EPT)], srcv)
    pltpu.sync_copy(dst_hbm.at[pl.ds(base_e, EPT)], dstv1)
    pltpu.sync_copy(ew_hbm.at[pl.ds(base_e, EPT)], ewv)
    pltpu.sync_copy(dis_hbm, disv)

    zero = jnp.zeros((L,), jnp.float32)

    def zb(i, _):
        for r in range(HQ // L):
            zbuf[i, pl.ds(r * L, L)] = zero
        return 0
    lax.fori_loop(0, 128, zb, 0)

    def cbody(i, _):
        s16 = srcv[pl.ds(i * L, L)]
        d16 = dstv1[pl.ds(i * L, L)]
        w16 = ewv[pl.ds(i * L, L)]
        c16 = plsc.load_gather(disv, [s16]) * w16 * plsc.load_gather(disv, [d16])
        ewv[pl.ds(i * L, L)] = c16
        srcv[pl.ds(i * L, L)] = s16 * 4 + c * 2
        dstv2[i, pl.ds(0, L)] = d16
        return 0
    lax.fori_loop(0, CH2, cbody, 0)

    for rnd in range(NRND):
        if rnd > 0:
            def inc(i, _):
                srcv[pl.ds(i * L, L)] = srcv[pl.ds(i * L, L)] + 1
                return 0
            lax.fori_loop(0, CH2, inc, 0)

        for j in range(5):
            pltpu.sync_copy(zbuf, acc.at[pl.ds(s * NRT + j * 128, 128)])
        plsc.subcore_barrier()

        def mbody(i, _):
            idx = srcv.at[pl.ds(i * L, L)]
            pltpu.async_copy(gr_hbm.at[idx], rows, gsem).wait()
            for j in range(L):
                cj = plsc.load_gather(ewv, [jnp.full((L,), i * L + j, jnp.int32)])
                for r in range(HQ // L):
                    rows[j, pl.ds(r * L, L)] = rows[j, pl.ds(r * L, L)] * cj
            pltpu.sync_copy(rows, acc.at[dstv2.at[i]], add=True)
            return 0
        lax.fori_loop(0, CH2, mbody, 0)
        plsc.subcore_barrier()

        pltpu.sync_copy(acc.at[pl.ds(s * NRT, NRT)],
                        out_hbm.at[c, rnd, pl.ds(s * NRT, NRT)])


R = 1000
GRID = N // R

def _leaky(x):
    return jnp.where(x >= 0, x, 0.01 * x)


def _w_spec(shape):
    return pl.BlockSpec(shape, lambda i: (0, 0))


_ROW128 = pl.BlockSpec((R, HID), lambda i: (i, 0))


def _tc1_body(x_ref, Wh1, bh1, Wl1, bl1, Wx1, bx1, Wg1, Wh2, bh2, Wl2, bl2,
              o11_ref, g1_ref, o12_ref):
    xb = x_ref[:]
    hp1 = jnp.dot(xb, Wh1[:], preferred_element_type=jnp.float32) + bh1[:]
    o11_ref[:] = _leaky(jnp.dot(hp1, Wl1[:], preferred_element_type=jnp.float32) + bl1[:])
    xp1 = jnp.dot(xb, Wx1[:], preferred_element_type=jnp.float32) + bx1[:]
    g1_ref[:] = jnp.dot(xp1, Wg1[:], preferred_element_type=jnp.float32)
    hp2 = jnp.dot(xb, Wh2[:], preferred_element_type=jnp.float32) + bh2[:]
    o12_ref[:] = _leaky(jnp.dot(hp2, Wl2[:], preferred_element_type=jnp.float32) + bl2[:])


def _tc1(x, Wh1, bh1, Wl1, bl1, Wx1, bx1, Wg1, Wh2, bh2, Wl2, bl2):
    f32 = jnp.float32
    return pl.pallas_call(
        _tc1_body,
        grid=(GRID,),
        in_specs=[_ROW128] + [_w_spec(a.shape) for a in
                              (Wh1, bh1, Wl1, bl1, Wx1, bx1, Wg1, Wh2, bh2, Wl2, bl2)],
        out_specs=[_ROW128, _ROW128, _ROW128],
        out_shape=[jax.ShapeDtypeStruct((N, HID), f32)] * 3,
    )(x, Wh1, bh1, Wl1, bl1, Wx1, bx1, Wg1, Wh2, bh2, Wl2, bl2)


def _tc2_body(s1, g1, dis, o11, Wx2a, Wx2b, bx2, Wg2, bg1, g2_ref):
    d = dis[:]
    inv = d * d
    o21 = _leaky(s1[:] + g1[:] * inv + bg1[:])
    x1a = jax.nn.sigmoid(o11[:])
    x1b = jax.nn.sigmoid(o21)
    xp2 = (jnp.dot(x1a, Wx2a[:], preferred_element_type=jnp.float32)
           + jnp.dot(x1b, Wx2b[:], preferred_element_type=jnp.float32) + bx2[:])
    g2_ref[:] = jnp.dot(xp2, Wg2[:], preferred_element_type=jnp.float32)


def _tc2(s1, g1, dis, o11, Wx2a, Wx2b, bx2, Wg2, bg1):
    f32 = jnp.float32
    dis_spec = pl.BlockSpec((R, 1), lambda i: (i, 0))
    return pl.pallas_call(
        _tc2_body,
        grid=(GRID,),
        in_specs=[_ROW128, _ROW128, dis_spec, _ROW128,
                  _w_spec(Wx2a.shape), _w_spec(Wx2b.shape), _w_spec(bx2.shape),
                  _w_spec(Wg2.shape), _w_spec(bg1.shape)],
        out_specs=[_ROW128],
        out_shape=[jax.ShapeDtypeStruct((N, HID), f32)],
    )(s1, g1, dis, o11, Wx2a, Wx2b, bx2, Wg2, bg1)[0]


def _tc3_body(s2, g2, dis, o12, Wca, Wcb, bc, bg2, out_ref):
    d = dis[:]
    inv = d * d
    o22 = _leaky(s2[:] + g2[:] * inv + bg2[:])
    a = jax.nn.sigmoid(o12[:])
    b = jax.nn.sigmoid(o22)
    logits = (jnp.dot(a, Wca[:], preferred_element_type=jnp.float32)
              + jnp.dot(b, Wcb[:], preferred_element_type=jnp.float32) + bc[:])
    m = jnp.max(logits, axis=-1, keepdims=True)
    lse = m + jnp.log(jnp.sum(jnp.exp(logits - m), axis=-1, keepdims=True))
    out_ref[:] = logits - lse


def _tc3(s2, g2, dis, o12, Wca, Wcb, bc, bg2):
    f32 = jnp.float32
    dis_spec = pl.BlockSpec((R, 1), lambda i: (i, 0))
    out_spec = pl.BlockSpec((R, NCLS), lambda i: (i, 0))
    return pl.pallas_call(
        _tc3_body,
        grid=(GRID,),
        in_specs=[_ROW128, _ROW128, dis_spec, _ROW128,
                  _w_spec(Wca.shape), _w_spec(Wcb.shape), _w_spec(bc.shape),
                  _w_spec(bg2.shape)],
        out_specs=[out_spec],
        out_shape=[jax.ShapeDtypeStruct((N, NCLS), f32)],
    )(s2, g2, dis, o12, Wca, Wcb, bc, bg2)[0]


def kernel(x, edge_index, edge_weight, Wh1, bh1, Wx1, bx1, Wl1, bl1, Wg1, bg1,
           Wh2, bh2, Wx2, bx2, Wl2, bl2, Wg2, bg2, Wc, bc):
    src = edge_index[0]
    dst = edge_index[1]

    b = lambda v: v.reshape(1, -1)

    degp = _deg_partial(dst, edge_weight).reshape(NS, NPAD)
    dis_pad = _tcdis(degp)
    o11, g1, o12 = _tc1(x, Wh1, b(bh1), Wl1, b(bl1), Wx1, b(bx1), Wg1,
                        Wh2, b(bh2), Wl2, b(bl2))
    def quarters(sp):
        return jnp.concatenate(
            [sp[0, 0, :N], sp[0, 1, :N], sp[1, 0, :N], sp[1, 1, :N]], axis=1)

    s1p = _spmv(g1.reshape(4 * N, HQ), src, dst, edge_weight, dis_pad)
    s1 = quarters(s1p)
    dis_col = dis_pad[:N].reshape(N, 1)
    g2 = _tc2(s1, g1, dis_col, o11,
              Wx2[:HID], Wx2[HID:], b(bx2), Wg2, b(bg1))
    s2p = _spmv(g2.reshape(4 * N, HQ), src, dst, edge_weight, dis_pad)
    s2 = quarters(s2p)
    out = _tc3(s2, g2, dis_col, o12,
               Wc[:HID], Wc[HID:], b(bc), b(bg2))
    return out


# speedup vs baseline: 12.8294x; 4.7466x over previous
import functools

import jax
import jax.numpy as jnp
from jax import lax
from jax.experimental import pallas as pl
from jax.experimental.pallas import tpu as pltpu
from jax.experimental.pallas import tpu_sc as plsc

N = 10000
E = 320000
FEAT = 128
HID = 128
NCLS = 40

NC = 2
NS = 16
L = 16
NW = NC * NS

NPAD = 10240
DEG_ROWS = NPAD // L
EPT_DEG = E // NS
EW_PER = E // NW
CH = EW_PER // L
ROWS_PER_TILE = N // NS

_MESH = plsc.VectorSubcoreMesh(
    core_axis_name="c", subcore_axis_name="s", num_cores=NC, num_subcores=NS)
_SC_PARAMS = pltpu.CompilerParams(
    needs_layout_passes=False, use_tc_tiling_on_sc=False)


def _rsqrt16(x):
    i = plsc.bitcast(x, jnp.int32)
    i = jnp.int32(0x5F3759DF) - lax.shift_right_logical(i, 1)
    y = plsc.bitcast(i, jnp.float32)
    for _ in range(3):
        y = y * (1.5 - 0.5 * x * y * y)
    return y


@functools.partial(
    pl.kernel,
    out_type=jax.ShapeDtypeStruct((NS * NPAD,), jnp.float32),
    mesh=_MESH,
    compiler_params=_SC_PARAMS,
    scratch_types=[
        pltpu.VMEM((EPT_DEG,), jnp.int32),
        pltpu.VMEM((EPT_DEG,), jnp.float32),
        pltpu.VMEM((NPAD,), jnp.float32),
    ],
)
def _deg_partial(dst_hbm, ew_hbm, degp_hbm, dstv, ewv, degv):
    c = lax.axis_index("c")
    s = lax.axis_index("s")

    @pl.when(c == 0)
    def _():
        base = s * EPT_DEG
        pltpu.sync_copy(dst_hbm.at[pl.ds(base, EPT_DEG)], dstv)
        pltpu.sync_copy(ew_hbm.at[pl.ds(base, EPT_DEG)], ewv)

        zero = jnp.zeros((L,), jnp.float32)

        def zbody(i, _):
            degv[pl.ds(i * L, L)] = zero
            return 0
        lax.fori_loop(0, NPAD // L, zbody, 0)

        def abody(i, _):
            d16 = dstv[pl.ds(i * L, L)]
            w16 = ewv[pl.ds(i * L, L)]
            plsc.addupdate_scatter(degv, [d16], w16)
            return 0
        lax.fori_loop(0, EPT_DEG // L, abody, 0)

        pltpu.sync_copy(degv, degp_hbm.at[pl.ds(s * NPAD, NPAD)])


def _tcdis_body(degp_ref, dis_ref):
    dis_ref[:] = lax.rsqrt(jnp.sum(degp_ref[:], axis=0, keepdims=True) + 1.0)


def _tcdis(degp):
    return pl.pallas_call(
        _tcdis_body,
        grid=(1,),
        in_specs=[pl.BlockSpec((NS, NPAD), lambda i: (0, 0))],
        out_specs=[pl.BlockSpec((1, NPAD), lambda i: (0, 0))],
        out_shape=[jax.ShapeDtypeStruct((1, NPAD), jnp.float32)],
    )(degp)[0].reshape(NPAD)


NRT = NPAD // NS
HQ = HID // 4
NRND = 2
EPT = E // NS
CH2 = EPT // L
KK = 80
CH3 = EPT // KK

@functools.partial(
    pl.kernel,
    out_type=jax.ShapeDtypeStruct((NC, NRND, NPAD, HQ), jnp.float32),
    mesh=_MESH,
    compiler_params=_SC_PARAMS,
    scratch_types=[
        pltpu.VMEM((EPT,), jnp.int32),
        pltpu.VMEM((EPT,), jnp.int32),
        pltpu.VMEM((CH3, KK), jnp.int32),
        pltpu.VMEM((EPT,), jnp.float32),
        pltpu.VMEM((NPAD,), jnp.float32),
        pltpu.VMEM((2, KK, HQ), jnp.float32),
        pltpu.VMEM((128, HQ), jnp.float32),
        pltpu.VMEM_SHARED((NPAD, HQ), jnp.float32),
        pltpu.SemaphoreType.DMA,
        pltpu.SemaphoreType.DMA,
    ],
)
def _spmv(gr_hbm, src_hbm, dst_hbm, ew_hbm, dis_hbm, out_hbm,
          srcv, dstv1, dstv2, ewv, disv, rows, zbuf, acc, gsem, ssem):
    c = lax.axis_index("c")
    s = lax.axis_index("s")
    base_e = s * EPT

    pltpu.sync_copy(src_hbm.at[pl.ds(base_e, EPT)], srcv)
    pltpu.sync_copy(dst_hbm.at[pl.ds(base_e, EPT)], dstv1)
    pltpu.sync_copy(ew_hbm.at[pl.ds(base_e, EPT)], ewv)
    pltpu.sync_copy(dis_hbm, disv)

    zero = jnp.zeros((L,), jnp.float32)

    def zb(i, _):
        for r in range(HQ // L):
            zbuf[i, pl.ds(r * L, L)] = zero
        return 0
    lax.fori_loop(0, 128, zb, 0)

    def cbody(i, _):
        for q in range(KK // L):
            e = i * KK + q * L
            s16 = srcv[pl.ds(e, L)]
            d16 = dstv1[pl.ds(e, L)]
            w16 = ewv[pl.ds(e, L)]
            c16 = (plsc.load_gather(disv, [s16]) * w16
                   * plsc.load_gather(disv, [d16]))
            ewv[pl.ds(e, L)] = c16
            srcv[pl.ds(e, L)] = s16 * 4 + c * 2
            dstv2[i, pl.ds(q * L, L)] = d16
        return 0
    lax.fori_loop(0, CH3, cbody, 0)

    def _scale(b, i):
        for j in range(KK):
            cj = plsc.load_gather(ewv, [jnp.full((L,), i * KK + j, jnp.int32)])
            for r in range(HQ // L):
                rows[b, j, pl.ds(r * L, L)] = rows[b, j, pl.ds(r * L, L)] * cj

    for rnd in range(NRND):
        if rnd > 0:
            def inc(i, _):
                srcv[pl.ds(i * L, L)] = srcv[pl.ds(i * L, L)] + 1
                return 0
            lax.fori_loop(0, CH2, inc, 0)

        for j in range(5):
            pltpu.sync_copy(zbuf, acc.at[pl.ds(s * NRT + j * 128, 128)])
        plsc.subcore_barrier()

        pltpu.async_copy(gr_hbm.at[srcv.at[pl.ds(0, KK)]], rows.at[0], gsem)

        def pair(k, _):
            for b in range(2):
                i = k * 2 + b

                @pl.when(i >= 2)
                def _():
                    pltpu.make_async_copy(
                        rows.at[b], acc.at[dstv2.at[i - 2]], ssem).wait()

                @pl.when(i + 1 < CH3)
                def _():
                    pltpu.async_copy(
                        gr_hbm.at[srcv.at[pl.ds((i + 1) * KK, KK)]],
                        rows.at[1 - b], gsem)

                pltpu.make_async_copy(
                    gr_hbm.at[srcv.at[pl.ds(i * KK, KK)]], rows.at[b],
                    gsem).wait()
                _scale(b, i)
                pltpu.async_copy(rows.at[b], acc.at[dstv2.at[i]], ssem,
                                 add=True)
            return 0
        lax.fori_loop(0, CH3 // 2, pair, 0)

        pltpu.make_async_copy(rows.at[0], acc.at[dstv2.at[CH3 - 2]], ssem).wait()
        pltpu.make_async_copy(rows.at[1], acc.at[dstv2.at[CH3 - 1]], ssem).wait()
        plsc.subcore_barrier()

        pltpu.sync_copy(acc.at[pl.ds(s * NRT, NRT)],
                        out_hbm.at[c, rnd, pl.ds(s * NRT, NRT)])


R = 1000
GRID = N // R

def _leaky(x):
    return jnp.where(x >= 0, x, 0.01 * x)


def _w_spec(shape):
    return pl.BlockSpec(shape, lambda i: (0, 0))


_ROW128 = pl.BlockSpec((R, HID), lambda i: (i, 0))


def _tc1_body(x_ref, Wh1, bh1, Wl1, bl1, Wx1, bx1, Wg1, Wh2, bh2, Wl2, bl2,
              o11_ref, g1_ref, o12_ref):
    xb = x_ref[:]
    hp1 = jnp.dot(xb, Wh1[:], preferred_element_type=jnp.float32) + bh1[:]
    o11_ref[:] = _leaky(jnp.dot(hp1, Wl1[:], preferred_element_type=jnp.float32) + bl1[:])
    xp1 = jnp.dot(xb, Wx1[:], preferred_element_type=jnp.float32) + bx1[:]
    g1_ref[:] = jnp.dot(xp1, Wg1[:], preferred_element_type=jnp.float32)
    hp2 = jnp.dot(xb, Wh2[:], preferred_element_type=jnp.float32) + bh2[:]
    o12_ref[:] = _leaky(jnp.dot(hp2, Wl2[:], preferred_element_type=jnp.float32) + bl2[:])


def _tc1(x, Wh1, bh1, Wl1, bl1, Wx1, bx1, Wg1, Wh2, bh2, Wl2, bl2):
    f32 = jnp.float32
    return pl.pallas_call(
        _tc1_body,
        grid=(GRID,),
        in_specs=[_ROW128] + [_w_spec(a.shape) for a in
                              (Wh1, bh1, Wl1, bl1, Wx1, bx1, Wg1, Wh2, bh2, Wl2, bl2)],
        out_specs=[_ROW128, _ROW128, _ROW128],
        out_shape=[jax.ShapeDtypeStruct((N, HID), f32)] * 3,
    )(x, Wh1, bh1, Wl1, bl1, Wx1, bx1, Wg1, Wh2, bh2, Wl2, bl2)


def _tc2_body(s1, g1, dis, o11, Wx2a, Wx2b, bx2, Wg2, bg1, g2_ref):
    d = dis[:]
    inv = d * d
    o21 = _leaky(s1[:] + g1[:] * inv + bg1[:])
    x1a = jax.nn.sigmoid(o11[:])
    x1b = jax.nn.sigmoid(o21)
    xp2 = (jnp.dot(x1a, Wx2a[:], preferred_element_type=jnp.float32)
           + jnp.dot(x1b, Wx2b[:], preferred_element_type=jnp.float32) + bx2[:])
    g2_ref[:] = jnp.dot(xp2, Wg2[:], preferred_element_type=jnp.float32)


def _tc2(s1, g1, dis, o11, Wx2a, Wx2b, bx2, Wg2, bg1):
    f32 = jnp.float32
    dis_spec = pl.BlockSpec((R, 1), lambda i: (i, 0))
    return pl.pallas_call(
        _tc2_body,
        grid=(GRID,),
        in_specs=[_ROW128, _ROW128, dis_spec, _ROW128,
                  _w_spec(Wx2a.shape), _w_spec(Wx2b.shape), _w_spec(bx2.shape),
                  _w_spec(Wg2.shape), _w_spec(bg1.shape)],
        out_specs=[_ROW128],
        out_shape=[jax.ShapeDtypeStruct((N, HID), f32)],
    )(s1, g1, dis, o11, Wx2a, Wx2b, bx2, Wg2, bg1)[0]


def _tc3_body(s2, g2, dis, o12, Wca, Wcb, bc, bg2, out_ref):
    d = dis[:]
    inv = d * d
    o22 = _leaky(s2[:] + g2[:] * inv + bg2[:])
    a = jax.nn.sigmoid(o12[:])
    b = jax.nn.sigmoid(o22)
    logits = (jnp.dot(a, Wca[:], preferred_element_type=jnp.float32)
              + jnp.dot(b, Wcb[:], preferred_element_type=jnp.float32) + bc[:])
    m = jnp.max(logits, axis=-1, keepdims=True)
    lse = m + jnp.log(jnp.sum(jnp.exp(logits - m), axis=-1, keepdims=True))
    out_ref[:] = logits - lse


def _tc3(s2, g2, dis, o12, Wca, Wcb, bc, bg2):
    f32 = jnp.float32
    dis_spec = pl.BlockSpec((R, 1), lambda i: (i, 0))
    out_spec = pl.BlockSpec((R, NCLS), lambda i: (i, 0))
    return pl.pallas_call(
        _tc3_body,
        grid=(GRID,),
        in_specs=[_ROW128, _ROW128, dis_spec, _ROW128,
                  _w_spec(Wca.shape), _w_spec(Wcb.shape), _w_spec(bc.shape),
                  _w_spec(bg2.shape)],
        out_specs=[out_spec],
        out_shape=[jax.ShapeDtypeStruct((N, NCLS), f32)],
    )(s2, g2, dis, o12, Wca, Wcb, bc, bg2)[0]


def kernel(x, edge_index, edge_weight, Wh1, bh1, Wx1, bx1, Wl1, bl1, Wg1, bg1,
           Wh2, bh2, Wx2, bx2, Wl2, bl2, Wg2, bg2, Wc, bc):
    src = edge_index[0]
    dst = edge_index[1]

    b = lambda v: v.reshape(1, -1)

    degp = _deg_partial(dst, edge_weight).reshape(NS, NPAD)
    dis_pad = _tcdis(degp)
    o11, g1, o12 = _tc1(x, Wh1, b(bh1), Wl1, b(bl1), Wx1, b(bx1), Wg1,
                        Wh2, b(bh2), Wl2, b(bl2))
    def quarters(sp):
        return jnp.concatenate(
            [sp[0, 0, :N], sp[0, 1, :N], sp[1, 0, :N], sp[1, 1, :N]], axis=1)

    s1p = _spmv(g1.reshape(4 * N, HQ), src, dst, edge_weight, dis_pad)
    s1 = quarters(s1p)
    dis_col = dis_pad[:N].reshape(N, 1)
    g2 = _tc2(s1, g1, dis_col, o11,
              Wx2[:HID], Wx2[HID:], b(bx2), Wg2, b(bg1))
    s2p = _spmv(g2.reshape(4 * N, HQ), src, dst, edge_weight, dis_pad)
    s2 = quarters(s2p)
    out = _tc3(s2, g2, dis_col, o12,
               Wc[:HID], Wc[HID:], b(bc), b(bg2))
    return out
